# baseline (speedup 1.0000x reference)
import functools

import numpy as np

import jax
import jax.numpy as jnp
from jax import lax
from jax.experimental import pallas as pl
from jax.experimental.pallas import tpu as pltpu
from jax.experimental.pallas import tpu_sc as plsc

_D = 512
_B = 4096
_L = 77
_LP = 80
_NC = 2
_NS = 16
_NW = _NC * _NS
_BPW = _B // _NW
_CH = _D // 16


def _sums_sc(ids, maskx, table):
    mesh = plsc.VectorSubcoreMesh(core_axis_name="c", subcore_axis_name="s")

    @functools.partial(
        pl.kernel,
        mesh=mesh,
        out_type=jax.ShapeDtypeStruct((_B, _D), jnp.float32),
        scratch_types=(
            [pltpu.VMEM((_LP,), jnp.int32)] * 4
            + [pltpu.VMEM((_LP, 16), jnp.float32)] * 4
            + [pltpu.VMEM((_LP, _D // 2), jnp.int32)] * 2
            + [pltpu.VMEM((_D,), jnp.float32)] * 2
            + [pltpu.SemaphoreType.DMA] * 8
        ),
    )
    def k(ids_hbm, maskx_hbm, table_hbm, out_hbm,
          idx0, idx1, idx2, idx3, wx0, wx1, wx2, wx3,
          rows_a, rows_b, acc_a, acc_b,
          si0, si1, si2, si3, sga, sgb, soa, sob):
        idxs = [idx0, idx1, idx2, idx3]
        wxs = [wx0, wx1, wx2, wx3]
        sis = [si0, si1, si2, si3]
        wid = lax.axis_index("s") * _NC + lax.axis_index("c")
        base = wid * _BPW

        def prefetch(row, q):
            pltpu.async_copy(ids_hbm.at[row], idxs[q], sis[q])
            pltpu.async_copy(maskx_hbm.at[row], wxs[q], sis[q])

        def wait_prefetch(q):
            pltpu.make_async_copy(ids_hbm.at[base], idxs[q], sis[q]).wait()
            pltpu.make_async_copy(maskx_hbm.at[base], wxs[q], sis[q]).wait()

        def fire(q, rows_v, sg):
            pltpu.async_copy(table_hbm.at[idxs[q]], rows_v, sg)

        def consume(row, q, rows_v, sg, acc_v, so, guard):
            pltpu.make_async_copy(table_hbm.at[idxs[q]], rows_v, sg).wait()
            wexp_v = wxs[q]
            if guard is None:
                pltpu.make_async_copy(acc_v, out_hbm.at[base], so).wait()
            else:
                @pl.when(guard)
                def _():
                    pltpu.make_async_copy(acc_v, out_hbm.at[base], so).wait()

            hi_mask = jnp.int32(-65536)
            for h in range(2):
                def lbody(l, accs, h=h):
                    w = wexp_v[l, :]
                    out = list(accs)
                    for g in range(8):
                        vi = rows_v[l, pl.ds((h * 8 + g) * 16, 16)]
                        lo = lax.bitcast_convert_type(
                            lax.shift_left(vi, 16), jnp.float32)
                        hi = lax.bitcast_convert_type(
                            lax.bitwise_and(vi, hi_mask), jnp.float32)
                        out[2 * g] = out[2 * g] + lo * w
                        out[2 * g + 1] = out[2 * g + 1] + hi * w
                    return tuple(out)

                init = tuple(jnp.zeros((16,), jnp.float32) for _ in range(16))
                accs = lax.fori_loop(0, _LP, lbody, init)
                for g in range(8):
                    acc_v[pl.ds((h * 8 + g) * 32, 16)] = accs[2 * g]
                    acc_v[pl.ds((h * 8 + g) * 32 + 16, 16)] = accs[2 * g + 1]
            pltpu.async_copy(acc_v, out_hbm.at[row], so)

        prefetch(base, 0)
        wait_prefetch(0)
        fire(0, rows_a, sga)
        prefetch(base + 1, 1)

        def body(i4, carry):
            r0 = base + 4 * i4
            wait_prefetch(1)
            fire(1, rows_b, sgb)
            prefetch(r0 + 2, 2)
            prefetch(r0 + 3, 3)
            consume(r0, 0, rows_a, sga, acc_a, soa, i4 > 0)
            wait_prefetch(2)
            fire(2, rows_a, sga)
            consume(r0 + 1, 1, rows_b, sgb, acc_b, sob, i4 > 0)
            prefetch(jnp.minimum(r0 + 4, jnp.int32(_B - 1)), 0)
            prefetch(jnp.minimum(r0 + 5, jnp.int32(_B - 1)), 1)
            wait_prefetch(3)
            fire(3, rows_b, sgb)
            consume(r0 + 2, 2, rows_a, sga, acc_a, soa, None)
            wait_prefetch(0)
            fire(0, rows_a, sga)
            consume(r0 + 3, 3, rows_b, sgb, acc_b, sob, None)
            return carry

        lax.fori_loop(0, _BPW // 4, body, jnp.int32(0))
        pltpu.make_async_copy(table_hbm.at[idx0], rows_a, sga).wait()
        wait_prefetch(1)
        pltpu.make_async_copy(acc_a, out_hbm.at[base], soa).wait()
        pltpu.make_async_copy(acc_b, out_hbm.at[base], sob).wait()

    return k(ids, maskx, table)


def _mm_body(s_ref, m_ref, w_ref, b_ref, o_ref):
    cnt = jnp.sum(m_ref[...].astype(jnp.float32), axis=1, keepdims=True)
    pooled = s_ref[...] / jnp.maximum(cnt, jnp.float32(1e-9))
    o_ref[...] = (
        lax.dot_general(pooled, w_ref[...],
                        (((1,), (1,)), ((), ())),
                        preferred_element_type=jnp.float32)
        + b_ref[0:1, :]
    )


def _project_tc(sums, mask_p, W, b):
    tb = 512
    b2 = jnp.tile(b[None, :], (8, 1))
    return pl.pallas_call(
        _mm_body,
        grid=(_B // tb,),
        in_specs=[
            pl.BlockSpec((tb, _D), lambda i: (i, 0)),
            pl.BlockSpec((tb, _LP), lambda i: (i, 0)),
            pl.BlockSpec((_D, _D), lambda i: (0, 0)),
            pl.BlockSpec((8, _D), lambda i: (0, 0)),
        ],
        out_specs=pl.BlockSpec((tb, _D), lambda i: (i, 0)),
        out_shape=jax.ShapeDtypeStruct((_B, _D), jnp.float32),
    )(sums, mask_p, W, b2)


_P = np.arange(_D)
_PERM = np.where(
    _P % 32 < 16,
    (_P // 32) * 32 + 2 * (_P % 32),
    (_P // 32) * 32 + 2 * (_P % 32 - 16) + 1,
)


def kernel(input_ids, attention_mask, emb_table, W, b):
    ids_p = jnp.pad(input_ids, ((0, 0), (0, _LP - _L)))
    mask_p = jnp.pad(attention_mask, ((0, 0), (0, _LP - _L)))
    maskx = jnp.broadcast_to(
        mask_p.astype(jnp.float32)[:, :, None], (_B, _LP, 16))
    table_i = lax.bitcast_convert_type(
        emb_table.astype(jnp.bfloat16).reshape(emb_table.shape[0], _D // 2, 2),
        jnp.int32)
    sums = _sums_sc(ids_p, maskx, table_i)
    out = _project_tc(sums, mask_p, W[:, _PERM], b)
    return out[:, None, :]


# speedup vs baseline: 2.5404x; 2.5404x over previous
import functools

import jax
import jax.numpy as jnp
from jax import lax
from jax.experimental import pallas as pl
from jax.experimental.pallas import tpu as pltpu
from jax.experimental.pallas import tpu_sc as plsc

_D = 512
_B = 4096
_L = 77
_LP = 80
_NC = 2
_NS = 16
_NW = _NC * _NS
_BPW = _B // _NW
_CH = _D // 16


def _sums_sc(ids, maskx, table):
    mesh = plsc.VectorSubcoreMesh(core_axis_name="c", subcore_axis_name="s")

    @functools.partial(
        pl.kernel,
        mesh=mesh,
        out_type=jax.ShapeDtypeStruct((_B, _D), jnp.float32),
        scratch_types=[
            pltpu.VMEM((_LP,), jnp.int32),
            pltpu.VMEM((_LP,), jnp.int32),
            pltpu.VMEM((_LP, 16), jnp.float32),
            pltpu.VMEM((_LP, 16), jnp.float32),
            pltpu.VMEM((_LP, _D), jnp.float32),
            pltpu.VMEM((_LP, _D), jnp.float32),
            pltpu.VMEM((_D,), jnp.float32),
            pltpu.SemaphoreType.DMA,
            pltpu.SemaphoreType.DMA,
        ],
    )
    def k(ids_hbm, maskx_hbm, table_hbm, out_hbm, idx_a, idx_b, wexp_a,
          wexp_b, rows_a, rows_b, acc_v, sem_a, sem_b):
        wid = lax.axis_index("s") * _NC + lax.axis_index("c")
        base = wid * _BPW

        def issue(row, idx_v, wexp_v, rows_v, sem):
            pltpu.sync_copy(ids_hbm.at[row], idx_v)
            pltpu.sync_copy(maskx_hbm.at[row], wexp_v)
            return pltpu.async_copy(table_hbm.at[idx_v], rows_v, sem)

        def consume(row, idx_v, wexp_v, rows_v, sem):
            pltpu.make_async_copy(table_hbm.at[idx_v], rows_v, sem).wait()

            def lbody(l, accs):
                w = wexp_v[l, :]
                return tuple(
                    accs[c] + rows_v[l, pl.ds(c * 16, 16)] * w
                    for c in range(_CH))

            init = tuple(jnp.zeros((16,), jnp.float32) for _ in range(_CH))
            accs = lax.fori_loop(0, _LP, lbody, init)
            for c in range(_CH):
                acc_v[pl.ds(c * 16, 16)] = accs[c]
            pltpu.sync_copy(acc_v, out_hbm.at[row])

        issue(base, idx_a, wexp_a, rows_a, sem_a)

        def body(i2, carry):
            row_a = base + 2 * i2
            row_b = row_a + 1
            issue(row_b, idx_b, wexp_b, rows_b, sem_b)
            consume(row_a, idx_a, wexp_a, rows_a, sem_a)
            row_n = jnp.minimum(row_a + 2, jnp.int32(_B - 1))
            issue(row_n, idx_a, wexp_a, rows_a, sem_a)
            consume(row_b, idx_b, wexp_b, rows_b, sem_b)
            return carry

        lax.fori_loop(0, _BPW // 2, body, jnp.int32(0))
        pltpu.make_async_copy(table_hbm.at[idx_a], rows_a, sem_a).wait()

    return k(ids, maskx, table)


def _mm_body(s_ref, m_ref, w_ref, b_ref, o_ref):
    cnt = jnp.sum(m_ref[...].astype(jnp.float32), axis=1, keepdims=True)
    pooled = s_ref[...] / jnp.maximum(cnt, jnp.float32(1e-9))
    o_ref[...] = (
        lax.dot_general(pooled, w_ref[...],
                        (((1,), (1,)), ((), ())),
                        preferred_element_type=jnp.float32)
        + b_ref[0:1, :]
    )


def _project_tc(sums, mask_p, W, b):
    tb = 512
    b2 = jnp.tile(b[None, :], (8, 1))
    return pl.pallas_call(
        _mm_body,
        grid=(_B // tb,),
        in_specs=[
            pl.BlockSpec((tb, _D), lambda i: (i, 0)),
            pl.BlockSpec((tb, _LP), lambda i: (i, 0)),
            pl.BlockSpec((_D, _D), lambda i: (0, 0)),
            pl.BlockSpec((8, _D), lambda i: (0, 0)),
        ],
        out_specs=pl.BlockSpec((tb, _D), lambda i: (i, 0)),
        out_shape=jax.ShapeDtypeStruct((_B, _D), jnp.float32),
    )(sums, mask_p, W, b2)


def kernel(input_ids, attention_mask, emb_table, W, b):
    ids_p = jnp.pad(input_ids, ((0, 0), (0, _LP - _L)))
    mask_p = jnp.pad(attention_mask, ((0, 0), (0, _LP - _L)))
    maskx = jnp.broadcast_to(
        mask_p.astype(jnp.float32)[:, :, None], (_B, _LP, 16))
    sums = _sums_sc(ids_p, maskx, emb_table)
    out = _project_tc(sums, mask_p, W, b)
    return out[:, None, :]
